# baseline (speedup 1.0000x reference)
import functools

import jax
import jax.numpy as jnp
import numpy as np
from jax import lax
from jax.experimental import pallas as pl
from jax.experimental.pallas import tpu as pltpu
from jax.experimental.pallas import tpu_sc as plsc

NUM_BUCKETS = 32
NUM_HEADS = 16
Q = 2048
K = 2048
TBL_W = 4096
T128_W = 3968
OFF = 2040
OFF128 = 1920


def _build128_kernel(embT_ref, out_ref, t8_ref):
    s = lax.broadcasted_iota(jnp.int32, (8, TBL_W), 0)
    x = lax.broadcasted_iota(jnp.int32, (8, TBL_W), 1)
    d = x - OFF - s
    n = -d
    ret = jnp.where(n < 0, 16, 0)
    n = jnp.abs(n)
    is_small = n < 8
    n_safe = jnp.maximum(n, 1).astype(jnp.float32)
    val = 8 + (jnp.log(n_safe / 8) / np.log(128 / 8) * 8).astype(jnp.int32)
    val = jnp.minimum(val, 15)
    b = ret + jnp.where(is_small, n, val)
    acc = jnp.zeros((8, TBL_W), jnp.float32)
    for j in range(NUM_BUCKETS):
        acc = acc + jnp.where(b == j, embT_ref[0, 0, j], 0.0)
    t8_ref[...] = acc
    for g in range(16):
        out_ref[0, 8 * g:8 * g + 8, :] = t8_ref[:, 120 - 8 * g:120 - 8 * g + T128_W]


def _sc_materialize(t128):
    mesh = plsc.VectorSubcoreMesh(core_axis_name="c", subcore_axis_name="s")

    @functools.partial(
        pl.kernel,
        out_type=jax.ShapeDtypeStruct((NUM_HEADS, Q, K), jnp.float32),
        mesh=mesh,
        scratch_types=[
            pltpu.VMEM_SHARED((2, 128, T128_W), jnp.float32),
            pltpu.SemaphoreType.DMA,
        ],
    )
    def k(t128_hbm, out_hbm, shared, sem):
        core = lax.axis_index("c")
        sid = lax.axis_index("s")
        for i in range(NUM_HEADS // 2):
            h = core * (NUM_HEADS // 2) + i
            buf = i % 2

            @pl.when(sid == 0)
            def _stage():
                pltpu.sync_copy(t128_hbm.at[h], shared.at[buf])

            plsc.subcore_barrier()
            start = OFF128 - 128 * sid
            pltpu.sync_copy(
                shared.at[buf, :, pl.ds(start, K)],
                out_hbm.at[h, pl.ds(128 * sid, 128), :],
            )
            plsc.subcore_barrier()

    return k(t128)


def kernel(embedding, query_length, key_length):
    del query_length, key_length
    embT = embedding.T.reshape(NUM_HEADS, 1, NUM_BUCKETS)
    t128 = pl.pallas_call(
        _build128_kernel,
        grid=(NUM_HEADS,),
        in_specs=[pl.BlockSpec((1, 1, NUM_BUCKETS), lambda h: (h, 0, 0))],
        out_specs=pl.BlockSpec((1, 128, T128_W), lambda h: (h, 0, 0)),
        out_shape=jax.ShapeDtypeStruct((NUM_HEADS, 128, T128_W), jnp.float32),
        scratch_shapes=[pltpu.VMEM((8, TBL_W), jnp.float32)],
    )(embT)
    out = _sc_materialize(t128)
    return out[None]


# speedup vs baseline: 2.5189x; 2.5189x over previous
import jax
import jax.numpy as jnp
import numpy as np
from jax.experimental import pallas as pl
from jax.experimental.pallas import tpu as pltpu

NUM_BUCKETS = 32
NUM_HEADS = 16
Q = 2048
K = 2048
TBL_W = 4096
T128_W = 3968
OFF = 2040
OFF128 = 1920


def _build_t128(embT_ref, h, t8_ref, t128_ref):
    s = jax.lax.broadcasted_iota(jnp.int32, (8, TBL_W), 0)
    x = jax.lax.broadcasted_iota(jnp.int32, (8, TBL_W), 1)
    d = x - OFF - s
    n = -d
    ret = jnp.where(n < 0, 16, 0)
    n = jnp.abs(n)
    is_small = n < 8
    n_safe = jnp.maximum(n, 1).astype(jnp.float32)
    val = 8 + (jnp.log(n_safe / 8) / np.log(128 / 8) * 8).astype(jnp.int32)
    val = jnp.minimum(val, 15)
    b = ret + jnp.where(is_small, n, val)
    acc = jnp.zeros((8, TBL_W), jnp.float32)
    for j in range(NUM_BUCKETS):
        acc = acc + jnp.where(b == j, embT_ref[h, 0, j], 0.0)
    t8_ref[...] = acc
    for g in range(16):
        t128_ref[8 * g:8 * g + 8, :] = t8_ref[:, 120 - 8 * g:120 - 8 * g + T128_W]


def _pbe_kernel(embT_ref, out_ref, t8_ref, t128_ref, sem):
    copies = [[], []]
    for h in range(NUM_HEADS):
        p = h % 2
        for c in copies[p]:
            c.wait()
        copies[p] = []
        _build_t128(embT_ref, h, t8_ref, t128_ref.at[p])
        for a in range(Q // 128):
            c = pltpu.make_async_copy(
                t128_ref.at[p, :, pl.ds(OFF128 - 128 * a, K)],
                out_ref.at[h, pl.ds(128 * a, 128), :],
                sem.at[p],
            )
            c.start()
            copies[p].append(c)
    for p in (0, 1):
        for c in copies[p]:
            c.wait()


def kernel(embedding, query_length, key_length):
    del query_length, key_length
    embT = embedding.T.reshape(NUM_HEADS, 1, NUM_BUCKETS)
    out = pl.pallas_call(
        _pbe_kernel,
        in_specs=[pl.BlockSpec(memory_space=pltpu.VMEM)],
        out_specs=pl.BlockSpec(memory_space=pl.ANY),
        out_shape=jax.ShapeDtypeStruct((NUM_HEADS, Q, K), jnp.float32),
        scratch_shapes=[
            pltpu.VMEM((8, TBL_W), jnp.float32),
            pltpu.VMEM((2, 128, T128_W), jnp.float32),
            pltpu.SemaphoreType.DMA((2,)),
        ],
    )(embT)
    return out[None]
